# baseline (speedup 1.0000x reference)
import functools
import jax
import jax.numpy as jnp
from jax import lax
from jax.experimental import pallas as pl
from jax.experimental.pallas import tpu as pltpu
from jax.experimental.pallas import tpu_sc as plsc

_NC, _NS = 2, 16
_NW = _NC * _NS


@functools.lru_cache(maxsize=None)
def _make_gather_max(R, D, Q, K):
    assert D % 16 == 0
    qpw = Q // _NW
    assert qpw * _NW == Q
    tile_q = max(1, min(qpw, 262144 // (K * D * 4)))
    while qpw % tile_q:
        tile_q -= 1
    n_sub = qpw // tile_q

    mesh = plsc.VectorSubcoreMesh(core_axis_name="c", subcore_axis_name="s")

    @functools.partial(
        pl.kernel,
        out_type=jax.ShapeDtypeStruct((Q, D), jnp.float32),
        mesh=mesh,
        scratch_types=[
            pltpu.VMEM((tile_q * K,), jnp.int32),
            pltpu.VMEM((tile_q * K, D), jnp.float32),
            pltpu.VMEM((tile_q, D), jnp.float32),
            pltpu.SemaphoreType.DMA,
        ],
        compiler_params=pltpu.CompilerParams(use_tc_tiling_on_sc=False),
    )
    def gather_max(table_hbm, idx_hbm, out_hbm, idx_v, rows_v, out_v, sem):
        wid = lax.axis_index("s") * _NC + lax.axis_index("c")
        base_q = wid * qpw

        def step(s, carry):
            q0 = base_q + s * tile_q
            pltpu.sync_copy(idx_hbm.at[pl.ds(q0 * K, tile_q * K)], idx_v)
            pltpu.async_copy(table_hbm.at[idx_v], rows_v, sem).wait()

            def qbody(q, c2):
                for c in range(D // 16):
                    sl = pl.ds(c * 16, 16)
                    acc = rows_v[q * K, sl]
                    for k in range(1, K):
                        acc = jnp.maximum(acc, rows_v[q * K + k, sl])
                    out_v[q, sl] = acc
                return c2

            lax.fori_loop(0, tile_q, qbody, 0, unroll=False)
            pltpu.sync_copy(out_v, out_hbm.at[pl.ds(q0, tile_q)])
            return carry

        lax.fori_loop(0, n_sub, step, 0, unroll=False)

    return gather_max


def _gather_max(table, idx_flat, K):
    R, D = table.shape
    Q = idx_flat.shape[0] // K
    return _make_gather_max(R, D, Q, K)(table, idx_flat)


def _lin(x, wb):
    return x @ wb[0] + wb[1]


def _quat_to_rotmat(q):
    q = q / jnp.linalg.norm(q, axis=1, keepdims=True)
    w, x, y, z = q[:, 0], q[:, 1], q[:, 2], q[:, 3]
    R = jnp.stack([
        1 - 2 * (y * y + z * z), 2 * (x * y - w * z), 2 * (x * z + w * y),
        2 * (x * y + w * z), 1 - 2 * (x * x + z * z), 2 * (y * z - w * x),
        2 * (x * z - w * y), 2 * (y * z + w * x), 1 - 2 * (x * x + y * y)],
        axis=1)
    return R.reshape(-1, 3, 3)


def _qstn(pos, p):
    x = jax.nn.relu(_lin(pos, p[0]))
    x = jax.nn.relu(_lin(x, p[1]))
    x = jax.nn.relu(_lin(x, p[2]))
    x = jnp.max(x, axis=1)
    x = jax.nn.relu(_lin(x, p[3]))
    x = jax.nn.relu(_lin(x, p[4]))
    x = _lin(x, p[5])
    x = x + jnp.array([1.0, 0.0, 0.0, 0.0], dtype=x.dtype)
    return _quat_to_rotmat(x)


@functools.lru_cache(maxsize=None)
def _make_knn_search(B, M, n, k):
    def body(q_ref, p_ref, d_ref, v_ref):
        q = q_ref[0]
        p = p_ref[0]
        dx = q[:, 0:1] - p[:, 0][None, :]
        dy = q[:, 1:2] - p[:, 1][None, :]
        dz = q[:, 2:3] - p[:, 2][None, :]
        d = dx * dx + dy * dy + dz * dz
        ri = lax.broadcasted_iota(jnp.int32, (M, n), 0)
        ci = lax.broadcasted_iota(jnp.int32, (M, n), 1)
        d = jnp.where(ri == ci, jnp.float32(jnp.inf), d)
        di = lax.bitcast_convert_type(d, jnp.int32)

        def it(_, lohi):
            lo, hi = lohi
            mid = lo + (hi - lo) // 2
            cnt = jnp.sum((di <= mid).astype(jnp.float32), axis=1, keepdims=True)
            ge = cnt >= k
            return jnp.where(ge, lo, mid + 1), jnp.where(ge, mid, hi)

        lo0 = jnp.zeros((M, 1), jnp.int32)
        hi0 = jnp.full((M, 1), 0x7F800000, jnp.int32)
        lo, hi = lax.fori_loop(0, 31, it, (lo0, hi0))
        d_ref[0] = di
        v_ref[0, 0] = hi[:, 0]

    return pl.pallas_call(
        body,
        grid=(B,),
        in_specs=[
            pl.BlockSpec((1, M, 3), lambda b: (b, 0, 0)),
            pl.BlockSpec((1, n, 3), lambda b: (b, 0, 0)),
        ],
        out_specs=[
            pl.BlockSpec((1, M, n), lambda b: (b, 0, 0)),
            pl.BlockSpec((1, 1, M), lambda b: (b, 0, 0)),
        ],
        out_shape=[
            jax.ShapeDtypeStruct((B, M, n), jnp.int32),
            jax.ShapeDtypeStruct((B, 1, M), jnp.int32),
        ],
    )


@functools.lru_cache(maxsize=None)
def _make_knn_compact(B, M, n, k):
    Q = B * M
    qpt = Q // _NW
    tpb = _NW // B
    chunk_q = min(qpt, 65536 // n)
    while qpt % chunk_q:
        chunk_q -= 1
    n_chunks = qpt // chunk_q

    mesh = plsc.VectorSubcoreMesh(core_axis_name="c", subcore_axis_name="s")

    @functools.partial(
        pl.kernel,
        out_type=jax.ShapeDtypeStruct((Q * k,), jnp.int32),
        mesh=mesh,
        scratch_types=[
            pltpu.VMEM((chunk_q, n), jnp.int32),
            pltpu.VMEM((qpt,), jnp.int32),
            pltpu.VMEM((qpt * k + 16,), jnp.int32),
        ],
        compiler_params=pltpu.CompilerParams(use_tc_tiling_on_sc=False,
                                             needs_layout_passes=False),
    )
    def compact(d_hbm, v_hbm, out_hbm, d_v, v_v, out_v, ):
        wid = lax.axis_index("s") * _NC + lax.axis_index("c")
        base_q = wid * qpt
        boff = (wid // tpb) * n
        pltpu.sync_copy(v_hbm.at[pl.ds(base_q, qpt)], v_v)
        lane = jax.lax.iota(jnp.int32, 16)

        def chunk_body(c, carry):
            q0 = c * chunk_q
            pltpu.sync_copy(d_hbm.at[pl.ds(base_q + q0, chunk_q)], d_v)

            def q_body(q, carry2):
                vs = plsc.load_gather(v_v, [jnp.full((16,), 0, jnp.int32) + (q0 + q)])

                def j_body(j, cursor):
                    v = d_v[q, pl.ds(j * 16, 16)]
                    mask = v <= vs
                    cnt = plsc.cumsum(mask.astype(jnp.int32))
                    gpos = cnt + (cursor - 1 + (q0 + q) * k)
                    val = lane + (j * 16 + boff)
                    plsc.store_scatter(out_v, [gpos], val, mask=mask)
                    return cursor + jnp.sum(mask.astype(jnp.int32))

                lax.fori_loop(0, n // 16, j_body, jnp.int32(0), unroll=False)
                return carry2

            lax.fori_loop(0, chunk_q, q_body, 0, unroll=False)
            return carry

        lax.fori_loop(0, n_chunks, chunk_body, 0, unroll=False)
        pltpu.sync_copy(out_v.at[pl.ds(0, qpt * k)],
                        out_hbm.at[pl.ds(base_q * k, qpt * k)])

    return compact


def _knn_idx_flat(pos_q, pos_p, k):
    B, M, _ = pos_q.shape
    n = pos_p.shape[1]
    d_i32, vstar = _make_knn_search(B, M, n, k)(pos_q, pos_p)
    out = _make_knn_compact(B, M, n, k)(d_i32.reshape(B * M, n),
                                        vstar.reshape(B * M))
    return out


def kernel(pos, knn_idx, knn_idx_l, params):
    B, N, _ = pos.shape
    BN = B * N

    trans = _qstn(pos, params["qstn"])
    pos = jnp.einsum('bnd,bde->bne', pos, trans)

    boffs = (jnp.arange(B, dtype=jnp.int32) * N)[:, None, None]
    idx_s = (knn_idx.astype(jnp.int32) + boffs)
    idx_s = jnp.concatenate([idx_s, idx_s], axis=-1)
    idx_l = (knn_idx_l.astype(jnp.int32) + boffs) + BN
    idx_lfe = jnp.concatenate(
        [idx_s.reshape(-1), idx_l.reshape(-1)], axis=0)

    x1 = pos.reshape(BN, 3)
    x2 = pos.reshape(BN, 3)
    for lvl in range(4):
        w1, b1 = params["enc1"][lvl]
        w2, b2 = params["enc2"][lvl]
        C = w1.shape[0] // 2
        a1 = x1 @ (w1[:C] - w1[C:]) + b1
        a2 = x2 @ (w2[:C] - w2[C:]) + b2
        t1 = x1 @ w1[C:]
        t2 = x2 @ w2[C:]
        table = jnp.concatenate([t1, t2], axis=0)
        table = jnp.pad(table, ((0, 0), (0, 8)))
        gmax = _gather_max(table, idx_lfe, 32)[:, :24]
        h1 = jax.nn.relu(a1 + gmax[:BN])
        h2 = jax.nn.relu(a2 + gmax[BN:])
        x1 = jnp.concatenate([x1, h1], axis=-1)
        x2 = jnp.concatenate([x2, h2], axis=-1)

    y1 = x1.reshape(B, N, -1)
    y2 = x2.reshape(B, N, -1)

    s = jax.nn.sigmoid(_lin(y1 + y2, params["att"]))
    y = s * y1 + (1 - s) * y2
    y = jax.nn.relu(_lin(y, params["c1"]))
    y = jax.nn.relu(_lin(y, params["c2"]))

    NUM_OUT = [512, 256, 128, 64]
    KNN_H1, KNN_H2 = 32, 16

    idx1 = _knn_idx_flat(pos[:, :NUM_OUT[0]], pos, KNN_H1)
    idx2 = _knn_idx_flat(pos[:, :NUM_OUT[1]], pos[:, :NUM_OUT[0]], KNN_H1)
    idx3 = _knn_idx_flat(pos[:, :NUM_OUT[2]], pos[:, :NUM_OUT[1]], KNN_H2)
    idx4 = _knn_idx_flat(pos[:, :NUM_OUT[2]], pos[:, :NUM_OUT[2]], KNN_H2)

    def hier(y, idxf, m, p, x_last, nf, K):
        Bb, Nsrc, Dd = y.shape
        agg = _gather_max(y.reshape(Bb * Nsrc, Dd), idxf, K)
        agg = agg.reshape(Bb, m, Dd)
        yc = y[:, :m]
        if nf != 1:
            agg = agg - yc
        f = jnp.concatenate([yc, agg], axis=-1)
        if x_last is not None:
            f = jnp.concatenate(
                [f, jnp.broadcast_to(x_last[:, None, :], (Bb, m, x_last.shape[1]))],
                axis=-1)
        y_new = jax.nn.relu(_lin(f, p[0]))
        g = jax.nn.relu(_lin(jnp.max(y_new, axis=1), p[1]))
        return y_new, g

    y, g1 = hier(y, idx1, NUM_OUT[0], params["s1"], None, 1, KNN_H1)
    y, g2 = hier(y, idx2, NUM_OUT[1], params["s2"], g1, 2, KNN_H1)
    y, g3 = hier(y, idx3, NUM_OUT[2], params["s3"], g2, 1, KNN_H2)
    y, g4 = hier(y, idx4, NUM_OUT[2], params["s4"], g3, 2, KNN_H2)

    y = jax.nn.relu(_lin(y, params["c3"])) + y
    y = jax.nn.relu(_lin(y, params["c4"]))
    yg = jax.nn.relu(_lin(y[:, :NUM_OUT[3]], params["cg"])) + y[:, :NUM_OUT[3]]
    y_g = jnp.max(yg, axis=1)
    h = jax.nn.relu(_lin(jnp.concatenate([g1, g2, g3, g4, y_g], axis=1),
                         params["mlp"][0]))
    patch_global = jax.nn.relu(_lin(h, params["mlp"][1]))
    return (jnp.transpose(y, (0, 2, 1)), trans, pos, patch_global)


# speedup vs baseline: 1.1702x; 1.1702x over previous
import functools
import jax
import jax.numpy as jnp
from jax import lax
from jax.experimental import pallas as pl
from jax.experimental.pallas import tpu as pltpu
from jax.experimental.pallas import tpu_sc as plsc

_NC, _NS = 2, 16
_NW = _NC * _NS


@functools.lru_cache(maxsize=None)
def _make_gather_max(R, D, Q, K):
    assert D % 16 == 0
    qpw = Q // _NW
    assert qpw * _NW == Q
    tile_q = max(1, min(qpw, 32768 // (K * D)))
    while qpw % tile_q:
        tile_q -= 1
    n_sub = qpw // tile_q
    if n_sub % 2:
        assert n_sub == 1 or tile_q % 2 == 0
        if n_sub == 1:
            tile_q //= 2
            n_sub = 2

    mesh = plsc.VectorSubcoreMesh(core_axis_name="c", subcore_axis_name="s")

    @functools.partial(
        pl.kernel,
        out_type=jax.ShapeDtypeStruct((Q, D), jnp.float32),
        mesh=mesh,
        scratch_types=[
            pltpu.VMEM((qpw * K,), jnp.int32),
            pltpu.VMEM((tile_q * K, D), jnp.float32),
            pltpu.VMEM((tile_q * K, D), jnp.float32),
            pltpu.VMEM((qpw, D), jnp.float32),
            pltpu.SemaphoreType.DMA,
            pltpu.SemaphoreType.DMA,
        ],
        compiler_params=pltpu.CompilerParams(use_tc_tiling_on_sc=False),
    )
    def gather_max(table_hbm, idx_hbm, out_hbm, idx_v, rows0, rows1, out_v,
                   sem0, sem1):
        wid = lax.axis_index("s") * _NC + lax.axis_index("c")
        base_q = wid * qpw
        pltpu.sync_copy(idx_hbm.at[pl.ds(base_q * K, qpw * K)], idx_v)
        bufs = (rows0, rows1)
        sems = (sem0, sem1)

        def start(s, b):
            pltpu.async_copy(
                table_hbm.at[idx_v.at[pl.ds(s * (tile_q * K), tile_q * K)]],
                bufs[b], sems[b])

        def wait(b):
            pltpu.make_async_copy(
                table_hbm.at[idx_v.at[pl.ds(0, tile_q * K)]],
                bufs[b], sems[b]).wait()

        def compute(s, b):
            rows = bufs[b]

            def qbody(q, c2):
                for c in range(D // 16):
                    sl = pl.ds(c * 16, 16)
                    acc = rows[q * K, sl]
                    for k in range(1, K):
                        acc = jnp.maximum(acc, rows[q * K + k, sl])
                    out_v[s * tile_q + q, sl] = acc
                return c2

            lax.fori_loop(0, tile_q, qbody, 0, unroll=False)

        start(0, 0)

        def pair(i, carry):
            s0 = i * 2
            start(s0 + 1, 1)
            wait(0)
            compute(s0, 0)

            @pl.when(s0 + 2 < n_sub)
            def _():
                start(s0 + 2, 0)

            wait(1)
            compute(s0 + 1, 1)
            return carry

        lax.fori_loop(0, n_sub // 2, pair, 0, unroll=False)
        pltpu.sync_copy(out_v, out_hbm.at[pl.ds(base_q, qpw)])

    return gather_max


def _gather_max(table, idx_flat, K):
    R, D = table.shape
    Q = idx_flat.shape[0] // K
    return _make_gather_max(R, D, Q, K)(table, idx_flat)


def _lin(x, wb):
    return x @ wb[0] + wb[1]


def _quat_to_rotmat(q):
    q = q / jnp.linalg.norm(q, axis=1, keepdims=True)
    w, x, y, z = q[:, 0], q[:, 1], q[:, 2], q[:, 3]
    R = jnp.stack([
        1 - 2 * (y * y + z * z), 2 * (x * y - w * z), 2 * (x * z + w * y),
        2 * (x * y + w * z), 1 - 2 * (x * x + z * z), 2 * (y * z - w * x),
        2 * (x * z - w * y), 2 * (y * z + w * x), 1 - 2 * (x * x + y * y)],
        axis=1)
    return R.reshape(-1, 3, 3)


def _qstn(pos, p):
    x = jax.nn.relu(_lin(pos, p[0]))
    x = jax.nn.relu(_lin(x, p[1]))
    x = jax.nn.relu(_lin(x, p[2]))
    x = jnp.max(x, axis=1)
    x = jax.nn.relu(_lin(x, p[3]))
    x = jax.nn.relu(_lin(x, p[4]))
    x = _lin(x, p[5])
    x = x + jnp.array([1.0, 0.0, 0.0, 0.0], dtype=x.dtype)
    return _quat_to_rotmat(x)


@functools.lru_cache(maxsize=None)
def _make_knn_search(B, M, n, k):
    def body(q_ref, p_ref, d_ref, v_ref):
        q = q_ref[0]
        p = p_ref[0]
        dx = q[:, 0:1] - p[:, 0][None, :]
        dy = q[:, 1:2] - p[:, 1][None, :]
        dz = q[:, 2:3] - p[:, 2][None, :]
        d = dx * dx + dy * dy + dz * dz
        ri = lax.broadcasted_iota(jnp.int32, (M, n), 0)
        ci = lax.broadcasted_iota(jnp.int32, (M, n), 1)
        d = jnp.where(ri == ci, jnp.float32(jnp.inf), d)
        di = lax.bitcast_convert_type(d, jnp.int32)

        def it(_, lohi):
            lo, hi = lohi
            mid = lo + (hi - lo) // 2
            cnt = jnp.sum((di <= mid).astype(jnp.float32), axis=1, keepdims=True)
            ge = cnt >= k
            return jnp.where(ge, lo, mid + 1), jnp.where(ge, mid, hi)

        lo0 = jnp.zeros((M, 1), jnp.int32)
        hi0 = jnp.full((M, 1), 0x7F800000, jnp.int32)
        lo, hi = lax.fori_loop(0, 31, it, (lo0, hi0))
        d_ref[0] = di
        v_ref[0, 0] = hi[:, 0]

    return pl.pallas_call(
        body,
        grid=(B,),
        in_specs=[
            pl.BlockSpec((1, M, 3), lambda b: (b, 0, 0)),
            pl.BlockSpec((1, n, 3), lambda b: (b, 0, 0)),
        ],
        out_specs=[
            pl.BlockSpec((1, M, n), lambda b: (b, 0, 0)),
            pl.BlockSpec((1, 1, M), lambda b: (b, 0, 0)),
        ],
        out_shape=[
            jax.ShapeDtypeStruct((B, M, n), jnp.int32),
            jax.ShapeDtypeStruct((B, 1, M), jnp.int32),
        ],
    )


@functools.lru_cache(maxsize=None)
def _make_knn_compact(B, M, n, k):
    Q = B * M
    qpt = Q // _NW
    tpb = _NW // B
    chunk_q = min(qpt, 65536 // n)
    while qpt % chunk_q:
        chunk_q -= 1
    n_chunks = qpt // chunk_q

    mesh = plsc.VectorSubcoreMesh(core_axis_name="c", subcore_axis_name="s")

    @functools.partial(
        pl.kernel,
        out_type=jax.ShapeDtypeStruct((Q * k,), jnp.int32),
        mesh=mesh,
        scratch_types=[
            pltpu.VMEM((chunk_q, n), jnp.int32),
            pltpu.VMEM((qpt,), jnp.int32),
            pltpu.VMEM((qpt * k + 16,), jnp.int32),
        ],
        compiler_params=pltpu.CompilerParams(use_tc_tiling_on_sc=False,
                                             needs_layout_passes=False),
    )
    def compact(d_hbm, v_hbm, out_hbm, d_v, v_v, out_v, ):
        wid = lax.axis_index("s") * _NC + lax.axis_index("c")
        base_q = wid * qpt
        boff = (wid // tpb) * n
        pltpu.sync_copy(v_hbm.at[pl.ds(base_q, qpt)], v_v)
        lane = jax.lax.iota(jnp.int32, 16)

        def chunk_body(c, carry):
            q0 = c * chunk_q
            pltpu.sync_copy(d_hbm.at[pl.ds(base_q + q0, chunk_q)], d_v)

            def q_body(q, carry2):
                vs = plsc.load_gather(v_v, [jnp.full((16,), 0, jnp.int32) + (q0 + q)])

                def j_body(j, cursor):
                    v = d_v[q, pl.ds(j * 16, 16)]
                    mask = v <= vs
                    cnt = plsc.cumsum(mask.astype(jnp.int32))
                    gpos = cnt + (cursor - 1 + (q0 + q) * k)
                    val = lane + (j * 16 + boff)
                    plsc.store_scatter(out_v, [gpos], val, mask=mask)
                    return cursor + jnp.sum(mask.astype(jnp.int32))

                lax.fori_loop(0, n // 16, j_body, jnp.int32(0), unroll=False)
                return carry2

            lax.fori_loop(0, chunk_q, q_body, 0, unroll=False)
            return carry

        lax.fori_loop(0, n_chunks, chunk_body, 0, unroll=False)
        pltpu.sync_copy(out_v.at[pl.ds(0, qpt * k)],
                        out_hbm.at[pl.ds(base_q * k, qpt * k)])

    return compact


def _knn_idx_flat(pos_q, pos_p, k):
    B, M, _ = pos_q.shape
    n = pos_p.shape[1]
    d_i32, vstar = _make_knn_search(B, M, n, k)(pos_q, pos_p)
    out = _make_knn_compact(B, M, n, k)(d_i32.reshape(B * M, n),
                                        vstar.reshape(B * M))
    return out


def kernel(pos, knn_idx, knn_idx_l, params):
    B, N, _ = pos.shape
    BN = B * N

    trans = _qstn(pos, params["qstn"])
    pos = jnp.einsum('bnd,bde->bne', pos, trans)

    boffs = (jnp.arange(B, dtype=jnp.int32) * N)[:, None, None]
    idx_s = (knn_idx.astype(jnp.int32) + boffs)
    idx_s = jnp.concatenate([idx_s, idx_s], axis=-1)
    idx_l = (knn_idx_l.astype(jnp.int32) + boffs) + BN
    idx_lfe = jnp.concatenate(
        [idx_s.reshape(-1), idx_l.reshape(-1)], axis=0)

    x1 = pos.reshape(BN, 3)
    x2 = pos.reshape(BN, 3)
    for lvl in range(4):
        w1, b1 = params["enc1"][lvl]
        w2, b2 = params["enc2"][lvl]
        C = w1.shape[0] // 2
        a1 = x1 @ (w1[:C] - w1[C:]) + b1
        a2 = x2 @ (w2[:C] - w2[C:]) + b2
        t1 = x1 @ w1[C:]
        t2 = x2 @ w2[C:]
        table = jnp.concatenate([t1, t2], axis=0)
        table = jnp.pad(table, ((0, 0), (0, 8)))
        gmax = _gather_max(table, idx_lfe, 32)[:, :24]
        h1 = jax.nn.relu(a1 + gmax[:BN])
        h2 = jax.nn.relu(a2 + gmax[BN:])
        x1 = jnp.concatenate([x1, h1], axis=-1)
        x2 = jnp.concatenate([x2, h2], axis=-1)

    y1 = x1.reshape(B, N, -1)
    y2 = x2.reshape(B, N, -1)

    s = jax.nn.sigmoid(_lin(y1 + y2, params["att"]))
    y = s * y1 + (1 - s) * y2
    y = jax.nn.relu(_lin(y, params["c1"]))
    y = jax.nn.relu(_lin(y, params["c2"]))

    NUM_OUT = [512, 256, 128, 64]
    KNN_H1, KNN_H2 = 32, 16

    idx1 = _knn_idx_flat(pos[:, :NUM_OUT[0]], pos, KNN_H1)
    idx2 = _knn_idx_flat(pos[:, :NUM_OUT[1]], pos[:, :NUM_OUT[0]], KNN_H1)
    idx3 = _knn_idx_flat(pos[:, :NUM_OUT[2]], pos[:, :NUM_OUT[1]], KNN_H2)
    idx4 = _knn_idx_flat(pos[:, :NUM_OUT[2]], pos[:, :NUM_OUT[2]], KNN_H2)

    def hier(y, idxf, m, p, x_last, nf, K):
        Bb, Nsrc, Dd = y.shape
        agg = _gather_max(y.reshape(Bb * Nsrc, Dd), idxf, K)
        agg = agg.reshape(Bb, m, Dd)
        yc = y[:, :m]
        if nf != 1:
            agg = agg - yc
        f = jnp.concatenate([yc, agg], axis=-1)
        if x_last is not None:
            f = jnp.concatenate(
                [f, jnp.broadcast_to(x_last[:, None, :], (Bb, m, x_last.shape[1]))],
                axis=-1)
        y_new = jax.nn.relu(_lin(f, p[0]))
        g = jax.nn.relu(_lin(jnp.max(y_new, axis=1), p[1]))
        return y_new, g

    y, g1 = hier(y, idx1, NUM_OUT[0], params["s1"], None, 1, KNN_H1)
    y, g2 = hier(y, idx2, NUM_OUT[1], params["s2"], g1, 2, KNN_H1)
    y, g3 = hier(y, idx3, NUM_OUT[2], params["s3"], g2, 1, KNN_H2)
    y, g4 = hier(y, idx4, NUM_OUT[2], params["s4"], g3, 2, KNN_H2)

    y = jax.nn.relu(_lin(y, params["c3"])) + y
    y = jax.nn.relu(_lin(y, params["c4"]))
    yg = jax.nn.relu(_lin(y[:, :NUM_OUT[3]], params["cg"])) + y[:, :NUM_OUT[3]]
    y_g = jnp.max(yg, axis=1)
    h = jax.nn.relu(_lin(jnp.concatenate([g1, g2, g3, g4, y_g], axis=1),
                         params["mlp"][0]))
    patch_global = jax.nn.relu(_lin(h, params["mlp"][1]))
    return (jnp.transpose(y, (0, 2, 1)), trans, pos, patch_global)
